# Initial kernel scaffold; baseline (speedup 1.0000x reference)
import jax
import jax.numpy as jnp
from jax.experimental import pallas as pl


def kernel(x, table):
    raise NotImplementedError("write your pallas kernel here")


# speedup vs baseline: 4.6630x; 4.6630x over previous
import functools

import jax
import jax.numpy as jnp
from jax import lax
from jax.experimental import pallas as pl
from jax.experimental.pallas import tpu as pltpu
from jax.experimental.pallas import tpu_sc as plsc

NC = 2
NS = 16
NW = NC * NS

IDX_W = 128
K = 4
CHUNK = K * IDX_W


@functools.partial(jax.jit, static_argnames=("n", "d"))
def _gather(x_flat2d, table, *, n, d):
    steps = n // (NW * CHUNK)
    rows_per_w = n // NW // IDX_W

    @functools.partial(
        pl.kernel,
        out_type=jax.ShapeDtypeStruct((n, d), jnp.float32),
        mesh=plsc.VectorSubcoreMesh(core_axis_name="c", subcore_axis_name="s"),
        scratch_types=[
            pltpu.VMEM((K, IDX_W), jnp.int32),
            pltpu.VMEM((CHUNK, d), jnp.float32),
            pltpu.SemaphoreType.DMA,
        ],
        compiler_params=pltpu.CompilerParams(use_tc_tiling_on_sc=False),
    )
    def body(x_hbm, table_hbm, out_hbm, idx_v, rows_v, sem):
        wid = lax.axis_index("s") * NC + lax.axis_index("c")

        def step(g, carry):
            r0 = wid * rows_per_w + g * K
            pltpu.sync_copy(x_hbm.at[pl.ds(r0, K)], idx_v)
            copies = [
                pltpu.async_copy(
                    table_hbm.at[idx_v.at[j]],
                    rows_v.at[pl.ds(j * IDX_W, IDX_W)],
                    sem,
                )
                for j in range(K)
            ]
            for c in copies:
                c.wait()
            pltpu.sync_copy(rows_v, out_hbm.at[pl.ds(r0 * IDX_W, CHUNK)])
            return carry

        lax.fori_loop(0, steps, step, 0)

    return body(x_flat2d, table)


def kernel(x, table):
    b, s = x.shape
    v, d = table.shape
    n = b * s
    x2d = x.reshape(n // IDX_W, IDX_W).astype(jnp.int32)
    out = _gather(x2d, table, n=n, d=d)
    return out.reshape(b, s, d)


# speedup vs baseline: 4.9583x; 1.0633x over previous
import functools

import jax
import jax.numpy as jnp
from jax import lax
from jax.experimental import pallas as pl
from jax.experimental.pallas import tpu as pltpu
from jax.experimental.pallas import tpu_sc as plsc

NC = 2
NS = 16
NW = NC * NS

IDX_W = 128
K = 4
CHUNK = K * IDX_W


@functools.partial(jax.jit, static_argnames=("n", "d"))
def _gather(x_flat2d, table, *, n, d):
    rows_per_w = n // NW // IDX_W
    pairs = rows_per_w // (2 * K)

    @functools.partial(
        pl.kernel,
        out_type=jax.ShapeDtypeStruct((n, d), jnp.float32),
        mesh=plsc.VectorSubcoreMesh(core_axis_name="c", subcore_axis_name="s"),
        scratch_types=[
            pltpu.VMEM((2 * K, IDX_W), jnp.int32),
            pltpu.VMEM((CHUNK, d), jnp.float32),
            pltpu.VMEM((CHUNK, d), jnp.float32),
            pltpu.SemaphoreType.DMA,
            pltpu.SemaphoreType.DMA,
        ],
        compiler_params=pltpu.CompilerParams(use_tc_tiling_on_sc=False),
    )
    def body(x_hbm, table_hbm, out_hbm, idx_v, rows0, rows1, gsem, osem):
        wid = lax.axis_index("s") * NC + lax.axis_index("c")

        def store_wait(rows_v):
            pltpu.make_async_copy(rows_v, out_hbm.at[pl.ds(0, CHUNK)], osem).wait()

        def pair_body(g, first):
            r0 = wid * rows_per_w + g * 2 * K
            pltpu.sync_copy(x_hbm.at[pl.ds(r0, 2 * K)], idx_v)
            if not first:
                store_wait(rows0)
            ga = [
                pltpu.async_copy(
                    table_hbm.at[idx_v.at[j]],
                    rows0.at[pl.ds(j * IDX_W, IDX_W)],
                    gsem,
                )
                for j in range(K)
            ]
            if not first:
                store_wait(rows1)
            for c in ga:
                c.wait()
            gb = [
                pltpu.async_copy(
                    table_hbm.at[idx_v.at[K + j]],
                    rows1.at[pl.ds(j * IDX_W, IDX_W)],
                    gsem,
                )
                for j in range(K)
            ]
            pltpu.async_copy(rows0, out_hbm.at[pl.ds(r0 * IDX_W, CHUNK)], osem)
            for c in gb:
                c.wait()
            pltpu.async_copy(
                rows1, out_hbm.at[pl.ds((r0 + K) * IDX_W, CHUNK)], osem
            )

        pair_body(0, True)

        def step(g, carry):
            pair_body(g, False)
            return carry

        lax.fori_loop(1, pairs, step, 0)
        store_wait(rows0)
        store_wait(rows1)

    return body(x_flat2d, table)


def kernel(x, table):
    b, s = x.shape
    v, d = table.shape
    n = b * s
    x2d = x.reshape(n // IDX_W, IDX_W).astype(jnp.int32)
    out = _gather(x2d, table, n=n, d=d)
    return out.reshape(b, s, d)


# speedup vs baseline: 4.9885x; 1.0061x over previous
import functools

import jax
import jax.numpy as jnp
from jax import lax
from jax.experimental import pallas as pl
from jax.experimental.pallas import tpu as pltpu
from jax.experimental.pallas import tpu_sc as plsc

NC = 2
NS = 16
NW = NC * NS

CB = 4
GW = 100


@functools.partial(jax.jit, static_argnames=("b", "s", "d"))
def _gather(x, table, *, b, s, d):
    batches_per_w = b // NW
    pairs = batches_per_w // (2 * CB)

    @functools.partial(
        pl.kernel,
        out_type=jax.ShapeDtypeStruct((b, s, d), jnp.float32),
        mesh=plsc.VectorSubcoreMesh(core_axis_name="c", subcore_axis_name="s"),
        scratch_types=[
            pltpu.VMEM((2 * CB, s), jnp.int32),
            pltpu.VMEM((CB, s, d), jnp.float32),
            pltpu.VMEM((CB, s, d), jnp.float32),
            pltpu.SemaphoreType.DMA,
            pltpu.SemaphoreType.DMA,
        ],
        compiler_params=pltpu.CompilerParams(use_tc_tiling_on_sc=False),
    )
    def body(x_hbm, table_hbm, out_hbm, idx_v, rows0, rows1, gsem, osem):
        wid = lax.axis_index("s") * NC + lax.axis_index("c")

        def fire_gathers(rows_v, half):
            return [
                pltpu.async_copy(
                    table_hbm.at[idx_v.at[half * CB + bb]],
                    rows_v.at[bb],
                    gsem,
                )
                for bb in range(CB)
            ]

        def store_wait(rows_v):
            pltpu.make_async_copy(rows_v, out_hbm.at[pl.ds(0, CB)], osem).wait()

        def pair_body(g, first):
            b0 = wid * batches_per_w + g * 2 * CB
            pltpu.sync_copy(x_hbm.at[pl.ds(b0, 2 * CB)], idx_v)
            if not first:
                store_wait(rows0)
            ga = fire_gathers(rows0, 0)
            if not first:
                store_wait(rows1)
            for c in ga:
                c.wait()
            gb = fire_gathers(rows1, 1)
            pltpu.async_copy(rows0, out_hbm.at[pl.ds(b0, CB)], osem)
            for c in gb:
                c.wait()
            pltpu.async_copy(rows1, out_hbm.at[pl.ds(b0 + CB, CB)], osem)

        pair_body(0, True)

        def step(g, carry):
            pair_body(g, False)
            return carry

        lax.fori_loop(1, pairs, step, 0)
        store_wait(rows0)
        store_wait(rows1)

    return body(x, table)


def kernel(x, table):
    b, s = x.shape
    v, d = table.shape
    return _gather(x.astype(jnp.int32), table, b=b, s=s, d=d)
